# Initial kernel scaffold; baseline (speedup 1.0000x reference)
import jax
import jax.numpy as jnp
from jax.experimental import pallas as pl


def kernel(x, edge_index, batch, W1, b1, W2, b2, W3, b3, centroids, std_scale, ac_temp, running_mean, running_var):
    raise NotImplementedError("write your pallas kernel here")


# speedup vs baseline: 11.4013x; 11.4013x over previous
import functools

import jax
import jax.numpy as jnp
from jax import lax
from jax.experimental import pallas as pl
from jax.experimental.pallas import tpu as pltpu
from jax.experimental.pallas import tpu_sc as plsc

_N = 10000
_E = 320000
_H = 128
_NC = 18
_G = 64

_NCORE = 2
_NSUB = 16
_NW = _NCORE * _NSUB
_EPT = _E // _NW
_K = 80
_NCHUNK = _EPT // _K
_NPAD = 10240
_RPT = _NPAD // _NSUB
_ZR = 128

_BM = 2000
_CH = 2000
_NBLK = _N // _CH

_mesh = plsc.VectorSubcoreMesh(core_axis_name="c", subcore_axis_name="s")


@functools.partial(
    pl.kernel,
    out_type=(jax.ShapeDtypeStruct((_NPAD, 16), jnp.float32),
              jax.ShapeDtypeStruct((_NPAD, 16), jnp.float32)),
    mesh=_mesh,
    scratch_types=[
        pltpu.VMEM_SHARED((_NPAD, 16), jnp.float32),
        pltpu.VMEM((_K,), jnp.int32),
        pltpu.VMEM((_K, 16), jnp.float32),
        pltpu.VMEM((_RPT, 16), jnp.float32),
    ],
)
def _deg_kernel(dst_hbm, c0_hbm, c1_hbm, acc, dstb, onesb, zb):
    cid = lax.axis_index("c")
    sid = lax.axis_index("s")
    row0 = sid * _RPT

    def orow(i, c):
        onesb[i, :] = jnp.ones((16,), jnp.float32)
        return c
    lax.fori_loop(0, _K, orow, 0)

    def zrow(i, c):
        zb[i, :] = jnp.zeros((16,), jnp.float32)
        return c
    lax.fori_loop(0, _RPT, zrow, 0)
    pltpu.sync_copy(zb, acc.at[pl.ds(row0, _RPT), :])
    plsc.subcore_barrier()

    base = (cid * _NSUB + sid) * _EPT

    def step(k, c):
        off = pl.multiple_of(base + k * _K, 8)
        pltpu.sync_copy(dst_hbm.at[pl.ds(off, _K)], dstb)
        pltpu.sync_copy(onesb, acc.at[dstb], add=True)
        return c
    lax.fori_loop(0, _NCHUNK, step, 0)
    plsc.subcore_barrier()

    @pl.when(cid == 0)
    def _():
        pltpu.sync_copy(acc.at[pl.ds(row0, _RPT), :], c0_hbm.at[pl.ds(row0, _RPT), :])

    @pl.when(cid == 1)
    def _():
        pltpu.sync_copy(acc.at[pl.ds(row0, _RPT), :], c1_hbm.at[pl.ds(row0, _RPT), :])


@functools.partial(
    pl.kernel,
    out_type=(jax.ShapeDtypeStruct((_NPAD, _H), jnp.float32),
              jax.ShapeDtypeStruct((_NPAD, _H), jnp.float32)),
    mesh=_mesh,
    scratch_types=[
        pltpu.VMEM_SHARED((_NPAD, _H), jnp.float32),
        pltpu.VMEM((_K,), jnp.int32),
        pltpu.VMEM((_K,), jnp.int32),
        pltpu.VMEM((_K, _H), jnp.float32),
        pltpu.VMEM((_ZR, _H), jnp.float32),
        pltpu.SemaphoreType.DMA,
    ],
)
def _agg_kernel(ap_hbm, src_hbm, dst_hbm, p0_hbm, p1_hbm, acc, srcb, dstb, rows, zb, sem):
    cid = lax.axis_index("c")
    sid = lax.axis_index("s")
    row0 = sid * _RPT

    def zrow(i, c):
        for j in range(8):
            zb[i, pl.ds(j * 16, 16)] = jnp.zeros((16,), jnp.float32)
        return c
    lax.fori_loop(0, _ZR, zrow, 0)
    for t in range(_RPT // _ZR):
        pltpu.sync_copy(zb, acc.at[pl.ds(row0 + t * _ZR, _ZR), :])
    plsc.subcore_barrier()

    base = (cid * _NSUB + sid) * _EPT

    def step(k, c):
        off = pl.multiple_of(base + k * _K, 8)
        pltpu.sync_copy(src_hbm.at[pl.ds(off, _K)], srcb)
        pltpu.sync_copy(dst_hbm.at[pl.ds(off, _K)], dstb)
        pltpu.async_copy(ap_hbm.at[srcb], rows, sem).wait()
        pltpu.sync_copy(rows, acc.at[dstb], add=True)
        return c
    lax.fori_loop(0, _NCHUNK, step, 0)
    plsc.subcore_barrier()

    @pl.when(cid == 0)
    def _():
        pltpu.sync_copy(acc.at[pl.ds(row0, _RPT), :], p0_hbm.at[pl.ds(row0, _RPT), :])

    @pl.when(cid == 1)
    def _():
        pltpu.sync_copy(acc.at[pl.ds(row0, _RPT), :], p1_hbm.at[pl.ds(row0, _RPT), :])


def _dinv_of(c0, c1):
    return lax.rsqrt(c0[:, 0:1] + c1[:, 0:1] + 1.0)


def _mm1_body(x_ref, w_ref, c0_ref, c1_ref, o_ref):
    dinv = _dinv_of(c0_ref[...], c1_ref[...])
    o_ref[...] = dinv * jnp.dot(x_ref[...], w_ref[...],
                                preferred_element_type=jnp.float32)


def _mm_first(x, W, c0, c1):
    return pl.pallas_call(
        _mm1_body,
        grid=(_N // _BM,),
        in_specs=[
            pl.BlockSpec((_BM, _H), lambda i: (i, 0)),
            pl.BlockSpec((_H, _H), lambda i: (0, 0)),
            pl.BlockSpec((_BM, 16), lambda i: (i, 0)),
            pl.BlockSpec((_BM, 16), lambda i: (i, 0)),
        ],
        out_specs=pl.BlockSpec((_BM, _H), lambda i: (i, 0)),
        out_shape=jax.ShapeDtypeStruct((_N, _H), jnp.float32),
    )(x, W, c0, c1)


def _mm_mid_body(p0_ref, p1_ref, ap_ref, c0_ref, c1_ref, b_ref, w_ref, o_ref):
    dinv = _dinv_of(c0_ref[...], c1_ref[...])
    h = dinv * (p0_ref[...] + p1_ref[...] + ap_ref[...]) + b_ref[...]
    h = jnp.maximum(h, 0.0)
    o_ref[...] = dinv * jnp.dot(h, w_ref[...], preferred_element_type=jnp.float32)


def _mm_mid(p0, p1, ap, c0, c1, b_row, W):
    return pl.pallas_call(
        _mm_mid_body,
        grid=(_N // _BM,),
        in_specs=[
            pl.BlockSpec((_BM, _H), lambda i: (i, 0)),
            pl.BlockSpec((_BM, _H), lambda i: (i, 0)),
            pl.BlockSpec((_BM, _H), lambda i: (i, 0)),
            pl.BlockSpec((_BM, 16), lambda i: (i, 0)),
            pl.BlockSpec((_BM, 16), lambda i: (i, 0)),
            pl.BlockSpec((1, _H), lambda i: (0, 0)),
            pl.BlockSpec((_H, _H), lambda i: (0, 0)),
        ],
        out_specs=pl.BlockSpec((_BM, _H), lambda i: (i, 0)),
        out_shape=jax.ShapeDtypeStruct((_N, _H), jnp.float32),
    )(p0, p1, ap, c0, c1, b_row, W)


def _pool_body(p0_ref, p1_ref, ap_ref, c0_ref, c1_ref, b_ref, bt_ref, cpad_ref,
               mx_ref, it_ref, o_ref, sums, cnt):
    i = pl.program_id(0)

    @pl.when(i == 0)
    def _():
        sums[...] = jnp.zeros_like(sums)
        cnt[...] = jnp.zeros_like(cnt)

    dinv = _dinv_of(c0_ref[...], c1_ref[...])
    h = dinv * (p0_ref[...] + p1_ref[...] + ap_ref[...]) + b_ref[...]
    bt = bt_ref[0]
    gi = lax.broadcasted_iota(jnp.int32, (_G, _CH), 0)
    s = (bt == gi).astype(jnp.float32)
    sums[...] += jnp.dot(s, h, preferred_element_type=jnp.float32)
    cnt[...] += jnp.sum(s, axis=1, keepdims=True)

    @pl.when(i == _NBLK - 1)
    def _():
        g = sums[...] / jnp.maximum(cnt[...], 1.0)
        cp = cpad_ref[...]
        cross = jnp.dot(g, cp, preferred_element_type=jnp.float32)
        cn2 = jnp.sum(cp * cp, axis=0, keepdims=True)
        gn2 = jnp.sum(g * g, axis=1, keepdims=True)
        d2 = jnp.maximum(gn2 + cn2 - 2.0 * cross, 0.0)
        dmin2 = jnp.minimum(d2[:, :64], d2[:, 64:])
        dist = jnp.sqrt(dmin2)
        lane64 = lax.broadcasted_iota(jnp.int32, (_G, 64), 1)
        md = jnp.min(jnp.where(lane64 < _NC, dist, 1e30), axis=1, keepdims=True)
        soft = 1.0 / (1.0 + jnp.exp(-(mx_ref[...] - md) * it_ref[...]))
        dist128 = jnp.concatenate([dist, dist], axis=1)
        lane = lax.broadcasted_iota(jnp.int32, (_G, _H), 1)
        o_ref[...] = jnp.where(lane < _NC, -dist128,
                               jnp.where(lane == _NC, soft, 0.0))


def _pool_head(p0, p1, ap, c0, c1, b_row, batch3, cpadT, mx_row, it_row):
    return pl.pallas_call(
        _pool_body,
        grid=(_NBLK,),
        in_specs=[
            pl.BlockSpec((_CH, _H), lambda i: (i, 0)),
            pl.BlockSpec((_CH, _H), lambda i: (i, 0)),
            pl.BlockSpec((_CH, _H), lambda i: (i, 0)),
            pl.BlockSpec((_CH, 16), lambda i: (i, 0)),
            pl.BlockSpec((_CH, 16), lambda i: (i, 0)),
            pl.BlockSpec((1, _H), lambda i: (0, 0)),
            pl.BlockSpec((1, 1, _CH), lambda i: (i, 0, 0)),
            pl.BlockSpec((_H, _H), lambda i: (0, 0)),
            pl.BlockSpec((1, _H), lambda i: (0, 0)),
            pl.BlockSpec((1, _H), lambda i: (0, 0)),
        ],
        out_specs=pl.BlockSpec((_G, _H), lambda i: (0, 0)),
        out_shape=jax.ShapeDtypeStruct((_G, _H), jnp.float32),
        scratch_shapes=[
            pltpu.VMEM((_G, _H), jnp.float32),
            pltpu.VMEM((_G, _H), jnp.float32),
        ],
    )(p0, p1, ap, c0, c1, b_row, batch3, cpadT, mx_row, it_row)


def kernel(x, edge_index, batch, W1, b1, W2, b2, W3, b3, centroids,
           std_scale, ac_temp, running_mean, running_var):
    src = edge_index[0]
    dst = edge_index[1]

    c0, c1 = _deg_kernel(dst)

    a1 = _mm_first(x, W1, c0, c1)
    q0, q1 = _agg_kernel(a1, src, dst)
    a2 = _mm_mid(q0, q1, a1, c0, c1, b1.reshape(1, _H), W2)
    q0, q1 = _agg_kernel(a2, src, dst)
    a3 = _mm_mid(q0, q1, a2, c0, c1, b2.reshape(1, _H), W3)
    q0, q1 = _agg_kernel(a3, src, dst)

    cpadT = (jnp.zeros((_H, 128), jnp.float32)
             .at[:, :_NC].set(centroids[:, 0, :].T)
             .at[:, 64:64 + _NC].set(centroids[:, 1, :].T))
    max_ac = running_mean + jnp.clip(jnp.maximum(std_scale, 0.0), 0.0, 5.0) * jnp.sqrt(running_var)
    mx_row = jnp.full((1, _H), max_ac, jnp.float32)
    it_row = jnp.full((1, _H), 1.0 / ac_temp, jnp.float32)
    batch3 = batch.reshape(_NBLK, 1, _CH)

    o = _pool_head(q0, q1, a3, c0, c1, b3.reshape(1, _H), batch3, cpadT,
                   mx_row, it_row)
    return o[:, :_NC], o[:, _NC]


# speedup vs baseline: 17.5970x; 1.5434x over previous
import functools

import jax
import jax.numpy as jnp
from jax import lax
from jax.experimental import pallas as pl
from jax.experimental.pallas import tpu as pltpu
from jax.experimental.pallas import tpu_sc as plsc

_N = 10000
_E = 320000
_H = 128
_NC = 18
_G = 64

_NCORE = 2
_NSUB = 16
_NW = _NCORE * _NSUB
_EPT = _E // _NW
_K = 80
_NCHUNK = _EPT // _K
_KA = 80
_NCA = _EPT // _KA
_NPAD = 10240
_RPT = _NPAD // _NSUB
_ZR = 128

_BM = 2000
_CH = 2000
_NBLK = _N // _CH

_mesh = plsc.VectorSubcoreMesh(core_axis_name="c", subcore_axis_name="s")


@functools.partial(
    pl.kernel,
    out_type=(jax.ShapeDtypeStruct((_NPAD, 16), jnp.float32),
              jax.ShapeDtypeStruct((_NPAD, 16), jnp.float32)),
    mesh=_mesh,
    scratch_types=[
        pltpu.VMEM_SHARED((_NPAD, 16), jnp.float32),
        pltpu.VMEM((_K,), jnp.int32),
        pltpu.VMEM((_K, 16), jnp.float32),
        pltpu.VMEM((_RPT, 16), jnp.float32),
    ],
)
def _deg_kernel(dst_hbm, c0_hbm, c1_hbm, acc, dstb, onesb, zb):
    cid = lax.axis_index("c")
    sid = lax.axis_index("s")
    row0 = sid * _RPT

    def orow(i, c):
        onesb[i, :] = jnp.ones((16,), jnp.float32)
        return c
    lax.fori_loop(0, _K, orow, 0)

    def zrow(i, c):
        zb[i, :] = jnp.zeros((16,), jnp.float32)
        return c
    lax.fori_loop(0, _RPT, zrow, 0)
    pltpu.sync_copy(zb, acc.at[pl.ds(row0, _RPT), :])
    plsc.subcore_barrier()

    base = (cid * _NSUB + sid) * _EPT

    def step(k, c):
        off = pl.multiple_of(base + k * _K, 8)
        pltpu.sync_copy(dst_hbm.at[pl.ds(off, _K)], dstb)
        pltpu.sync_copy(onesb, acc.at[dstb], add=True)
        return c
    lax.fori_loop(0, _NCHUNK, step, 0)
    plsc.subcore_barrier()

    @pl.when(cid == 0)
    def _():
        pltpu.sync_copy(acc.at[pl.ds(row0, _RPT), :], c0_hbm.at[pl.ds(row0, _RPT), :])

    @pl.when(cid == 1)
    def _():
        pltpu.sync_copy(acc.at[pl.ds(row0, _RPT), :], c1_hbm.at[pl.ds(row0, _RPT), :])


@functools.partial(
    pl.kernel,
    out_type=(jax.ShapeDtypeStruct((_NPAD, _H), jnp.float32),
              jax.ShapeDtypeStruct((_NPAD, _H), jnp.float32)),
    mesh=_mesh,
    scratch_types=[
        pltpu.VMEM_SHARED((_NPAD, _H), jnp.float32),
        pltpu.VMEM((_KA,), jnp.int32),
        pltpu.VMEM((_KA,), jnp.int32),
        pltpu.VMEM((_KA,), jnp.int32),
        pltpu.VMEM((_KA,), jnp.int32),
        pltpu.VMEM((_KA, _H), jnp.float32),
        pltpu.VMEM((_KA, _H), jnp.float32),
        pltpu.VMEM((_ZR, _H), jnp.float32),
        pltpu.SemaphoreType.DMA,
        pltpu.SemaphoreType.DMA,
    ],
)
def _agg_kernel(ap_hbm, src_hbm, dst_hbm, p0_hbm, p1_hbm, acc,
                ebs0, ebd0, ebs1, ebd1, rows0, rows1,
                zb, sem0, sem1):
    cid = lax.axis_index("c")
    sid = lax.axis_index("s")
    row0 = sid * _RPT

    def zrow(i, c):
        for j in range(8):
            zb[i, pl.ds(j * 16, 16)] = jnp.zeros((16,), jnp.float32)
        return c
    lax.fori_loop(0, _ZR, zrow, 0)
    for t in range(_RPT // _ZR):
        pltpu.sync_copy(zb, acc.at[pl.ds(row0 + t * _ZR, _ZR), :])
    plsc.subcore_barrier()

    base = (cid * _NSUB + sid) * _EPT

    off0 = pl.multiple_of(base, 8)
    pltpu.sync_copy(src_hbm.at[pl.ds(off0, _KA)], ebs0)
    pltpu.sync_copy(dst_hbm.at[pl.ds(off0, _KA)], ebd0)
    pltpu.async_copy(ap_hbm.at[ebs0], rows0, sem0)

    def pair(j, c):
        off1 = pl.multiple_of(base + (2 * j + 1) * _KA, 8)
        pltpu.sync_copy(src_hbm.at[pl.ds(off1, _KA)], ebs1)
        pltpu.sync_copy(dst_hbm.at[pl.ds(off1, _KA)], ebd1)
        pltpu.async_copy(ap_hbm.at[ebs1], rows1, sem1)
        pltpu.make_async_copy(ap_hbm.at[ebs0], rows0, sem0).wait()
        pltpu.sync_copy(rows0, acc.at[ebd0], add=True)

        off2 = pl.multiple_of(base + (2 * j + 2) * _KA, 8)
        pltpu.sync_copy(src_hbm.at[pl.ds(off2, _KA)], ebs0)
        pltpu.sync_copy(dst_hbm.at[pl.ds(off2, _KA)], ebd0)
        pltpu.async_copy(ap_hbm.at[ebs0], rows0, sem0)

        pltpu.make_async_copy(ap_hbm.at[ebs1], rows1, sem1).wait()
        pltpu.sync_copy(rows1, acc.at[ebd1], add=True)
        return c
    lax.fori_loop(0, (_NCA - 1) // 2, pair, 0)

    pltpu.make_async_copy(ap_hbm.at[ebs0], rows0, sem0).wait()
    pltpu.sync_copy(rows0, acc.at[ebd0], add=True)
    plsc.subcore_barrier()

    @pl.when(cid == 0)
    def _():
        pltpu.sync_copy(acc.at[pl.ds(row0, _RPT), :], p0_hbm.at[pl.ds(row0, _RPT), :])

    @pl.when(cid == 1)
    def _():
        pltpu.sync_copy(acc.at[pl.ds(row0, _RPT), :], p1_hbm.at[pl.ds(row0, _RPT), :])


def _dinv_of(c0, c1):
    return lax.rsqrt(c0[:, 0:1] + c1[:, 0:1] + 1.0)


def _mm1_body(x_ref, w_ref, c0_ref, c1_ref, o_ref):
    dinv = _dinv_of(c0_ref[...], c1_ref[...])
    o_ref[...] = dinv * jnp.dot(x_ref[...], w_ref[...],
                                preferred_element_type=jnp.float32)


def _mm_first(x, W, c0, c1):
    return pl.pallas_call(
        _mm1_body,
        grid=(_N // _BM,),
        in_specs=[
            pl.BlockSpec((_BM, _H), lambda i: (i, 0)),
            pl.BlockSpec((_H, _H), lambda i: (0, 0)),
            pl.BlockSpec((_BM, 16), lambda i: (i, 0)),
            pl.BlockSpec((_BM, 16), lambda i: (i, 0)),
        ],
        out_specs=pl.BlockSpec((_BM, _H), lambda i: (i, 0)),
        out_shape=jax.ShapeDtypeStruct((_N, _H), jnp.float32),
    )(x, W, c0, c1)


def _mm_mid_body(p0_ref, p1_ref, ap_ref, c0_ref, c1_ref, b_ref, w_ref, o_ref):
    dinv = _dinv_of(c0_ref[...], c1_ref[...])
    h = dinv * (p0_ref[...] + p1_ref[...] + ap_ref[...]) + b_ref[...]
    h = jnp.maximum(h, 0.0)
    o_ref[...] = dinv * jnp.dot(h, w_ref[...], preferred_element_type=jnp.float32)


def _mm_mid(p0, p1, ap, c0, c1, b_row, W):
    return pl.pallas_call(
        _mm_mid_body,
        grid=(_N // _BM,),
        in_specs=[
            pl.BlockSpec((_BM, _H), lambda i: (i, 0)),
            pl.BlockSpec((_BM, _H), lambda i: (i, 0)),
            pl.BlockSpec((_BM, _H), lambda i: (i, 0)),
            pl.BlockSpec((_BM, 16), lambda i: (i, 0)),
            pl.BlockSpec((_BM, 16), lambda i: (i, 0)),
            pl.BlockSpec((1, _H), lambda i: (0, 0)),
            pl.BlockSpec((_H, _H), lambda i: (0, 0)),
        ],
        out_specs=pl.BlockSpec((_BM, _H), lambda i: (i, 0)),
        out_shape=jax.ShapeDtypeStruct((_N, _H), jnp.float32),
    )(p0, p1, ap, c0, c1, b_row, W)


def _pool_body(p0_ref, p1_ref, ap_ref, c0_ref, c1_ref, b_ref, bt_ref, cpad_ref,
               mx_ref, it_ref, o_ref, sums, cnt):
    i = pl.program_id(0)

    @pl.when(i == 0)
    def _():
        sums[...] = jnp.zeros_like(sums)
        cnt[...] = jnp.zeros_like(cnt)

    dinv = _dinv_of(c0_ref[...], c1_ref[...])
    h = dinv * (p0_ref[...] + p1_ref[...] + ap_ref[...]) + b_ref[...]
    bt = bt_ref[0]
    gi = lax.broadcasted_iota(jnp.int32, (_G, _CH), 0)
    s = (bt == gi).astype(jnp.float32)
    sums[...] += jnp.dot(s, h, preferred_element_type=jnp.float32)
    cnt[...] += jnp.sum(s, axis=1, keepdims=True)

    @pl.when(i == _NBLK - 1)
    def _():
        g = sums[...] / jnp.maximum(cnt[...], 1.0)
        cp = cpad_ref[...]
        cross = jnp.dot(g, cp, preferred_element_type=jnp.float32)
        cn2 = jnp.sum(cp * cp, axis=0, keepdims=True)
        gn2 = jnp.sum(g * g, axis=1, keepdims=True)
        d2 = jnp.maximum(gn2 + cn2 - 2.0 * cross, 0.0)
        dmin2 = jnp.minimum(d2[:, :64], d2[:, 64:])
        dist = jnp.sqrt(dmin2)
        lane64 = lax.broadcasted_iota(jnp.int32, (_G, 64), 1)
        md = jnp.min(jnp.where(lane64 < _NC, dist, 1e30), axis=1, keepdims=True)
        soft = 1.0 / (1.0 + jnp.exp(-(mx_ref[...] - md) * it_ref[...]))
        dist128 = jnp.concatenate([dist, dist], axis=1)
        lane = lax.broadcasted_iota(jnp.int32, (_G, _H), 1)
        o_ref[...] = jnp.where(lane < _NC, -dist128,
                               jnp.where(lane == _NC, soft, 0.0))


def _pool_head(p0, p1, ap, c0, c1, b_row, batch3, cpadT, mx_row, it_row):
    return pl.pallas_call(
        _pool_body,
        grid=(_NBLK,),
        in_specs=[
            pl.BlockSpec((_CH, _H), lambda i: (i, 0)),
            pl.BlockSpec((_CH, _H), lambda i: (i, 0)),
            pl.BlockSpec((_CH, _H), lambda i: (i, 0)),
            pl.BlockSpec((_CH, 16), lambda i: (i, 0)),
            pl.BlockSpec((_CH, 16), lambda i: (i, 0)),
            pl.BlockSpec((1, _H), lambda i: (0, 0)),
            pl.BlockSpec((1, 1, _CH), lambda i: (i, 0, 0)),
            pl.BlockSpec((_H, _H), lambda i: (0, 0)),
            pl.BlockSpec((1, _H), lambda i: (0, 0)),
            pl.BlockSpec((1, _H), lambda i: (0, 0)),
        ],
        out_specs=pl.BlockSpec((_G, _H), lambda i: (0, 0)),
        out_shape=jax.ShapeDtypeStruct((_G, _H), jnp.float32),
        scratch_shapes=[
            pltpu.VMEM((_G, _H), jnp.float32),
            pltpu.VMEM((_G, _H), jnp.float32),
        ],
    )(p0, p1, ap, c0, c1, b_row, batch3, cpadT, mx_row, it_row)


def kernel(x, edge_index, batch, W1, b1, W2, b2, W3, b3, centroids,
           std_scale, ac_temp, running_mean, running_var):
    src = edge_index[0]
    dst = edge_index[1]

    c0, c1 = _deg_kernel(dst)

    a1 = _mm_first(x, W1, c0, c1)
    q0, q1 = _agg_kernel(a1, src, dst)
    a2 = _mm_mid(q0, q1, a1, c0, c1, b1.reshape(1, _H), W2)
    q0, q1 = _agg_kernel(a2, src, dst)
    a3 = _mm_mid(q0, q1, a2, c0, c1, b2.reshape(1, _H), W3)
    q0, q1 = _agg_kernel(a3, src, dst)

    cpadT = (jnp.zeros((_H, 128), jnp.float32)
             .at[:, :_NC].set(centroids[:, 0, :].T)
             .at[:, 64:64 + _NC].set(centroids[:, 1, :].T))
    max_ac = running_mean + jnp.clip(jnp.maximum(std_scale, 0.0), 0.0, 5.0) * jnp.sqrt(running_var)
    mx_row = jnp.full((1, _H), max_ac, jnp.float32)
    it_row = jnp.full((1, _H), 1.0 / ac_temp, jnp.float32)
    batch3 = batch.reshape(_NBLK, 1, _CH)

    o = _pool_head(q0, q1, a3, c0, c1, b3.reshape(1, _H), batch3, cpadT,
                   mx_row, it_row)
    return o[:, :_NC], o[:, _NC]
